# baseline (speedup 1.0000x reference)
import functools

import jax
import jax.numpy as jnp
from jax import lax
from jax.experimental import pallas as pl
from jax.experimental.pallas import tpu as pltpu
from jax.experimental.pallas import tpu_sc as plsc

N = 50000
E = 800000
D_IN = 16
H = 128
BATCH = 128
NBAT = E // BATCH
RPT = 3128
N_PAD = 16 * RPT

_f32 = jnp.float32
_i32 = jnp.int32


def _scale_rows(wv_b, k, rows, width):
    nv = width // 16

    def group_body(g, carry):
        wvec = wv_b[k, pl.ds(pl.multiple_of(g * 16, 16), 16)]
        for kk in range(16):
            j = g * 16 + kk
            spl = jnp.full((16,), wvec[kk], dtype=_f32)
            for v in range(nv):
                rows[j, pl.ds(16 * v, 16)] = rows[j, pl.ds(16 * v, 16)] * spl
        return carry

    lax.fori_loop(0, BATCH // 16, group_body, 0)


def _scale_convert(wv_b, k, rows_i, rows32):
    def group_body(g, carry):
        wvec = wv_b[k, pl.ds(pl.multiple_of(g * 16, 16), 16)]
        for kk in range(16):
            j = g * 16 + kk
            w = rows_i[j]
            lo = lax.bitcast_convert_type(jnp.left_shift(w, 16), _f32)
            hi = lax.bitcast_convert_type(
                jnp.bitwise_and(w, jnp.int32(-65536)), _f32)
            spl = jnp.full((16,), wvec[kk], dtype=_f32)
            rows32[j, pl.ds(0, 16)] = lo * spl
            rows32[j, pl.ds(16, 16)] = hi * spl
        return carry

    lax.fori_loop(0, BATCH // 16, group_body, 0)


def _edge_pipeline(tab_h, src_h, dst_h, w_h, start, pn, sidx_b, didx_b,
                   wv_b, bufs, semi, semg, sems, acc, width, extra=None,
                   stages=None):
    def idx_load(t, k):
        eb = pl.multiple_of((start + t) * BATCH, BATCH)
        pltpu.async_copy(src_h.at[pl.ds(eb, BATCH)], sidx_b.at[k], semi[k])
        pltpu.async_copy(dst_h.at[pl.ds(eb, BATCH)], didx_b.at[k], semi[k])
        pltpu.async_copy(w_h.at[pl.ds(eb, BATCH)], wv_b.at[k], semi[k])

    def wait_idx(k):
        pltpu.make_async_copy(src_h.at[pl.ds(0, BATCH)], sidx_b.at[k],
                              semi[k]).wait()
        pltpu.make_async_copy(dst_h.at[pl.ds(0, BATCH)], didx_b.at[k],
                              semi[k]).wait()
        pltpu.make_async_copy(w_h.at[pl.ds(0, BATCH)], wv_b.at[k],
                              semi[k]).wait()

    def gather(k):
        pltpu.async_copy(tab_h.at[sidx_b.at[k]], bufs[k], semg[k])

    def wait_gather(k):
        pltpu.make_async_copy(tab_h.at[sidx_b.at[k]], bufs[k], semg[k]).wait()

    vals = bufs if stages is None else stages

    def wait_scatter(k):
        pltpu.make_async_copy(vals[k], acc.at[didx_b.at[k]], sems[k]).wait()

    idx_load(0, 0)
    idx_load(1, 1)
    wait_idx(0)
    gather(0)

    def batch_body(t, carry):
        for k in range(3):
            @pl.when(lax.rem(t, 3) == k)
            def _(k=k):
                k1 = (k + 1) % 3
                k2 = (k + 2) % 3

                @pl.when(t + 1 < pn)
                def _():
                    wait_idx(k1)
                    gather(k1)

                wait_gather(k)
                if stages is None:
                    _scale_rows(wv_b, k, bufs[k], width)
                else:
                    _scale_convert(wv_b, k, bufs[k], stages[k])
                pltpu.async_copy(vals[k], acc.at[didx_b.at[k]], sems[k],
                                 add=True)
                if extra is not None:
                    extra(t, k)

                @pl.when(t >= 1)
                def _():
                    wait_scatter(k2)

                @pl.when(t + 2 < pn)
                def _():
                    idx_load(t + 2, k2)
        return carry

    lax.fori_loop(0, pn, batch_body, 0)
    td = pn - 1
    for k in range(3):
        @pl.when(lax.rem(td, 3) == k)
        def _(k=k):
            wait_scatter(k)


def _make_sc_layer1():
    mesh = plsc.VectorSubcoreMesh(core_axis_name="c", subcore_axis_name="s")

    @functools.partial(
        pl.kernel,
        mesh=mesh,
        out_type=[
            jax.ShapeDtypeStruct((2, N_PAD, D_IN), _f32),
            jax.ShapeDtypeStruct((2, N_PAD, D_IN), _f32),
        ],
        scratch_types=[
            pltpu.VMEM((3, BATCH), _i32),
            pltpu.VMEM((3, BATCH), _i32),
            pltpu.VMEM((3, BATCH), _f32),
            pltpu.VMEM((BATCH, D_IN), _f32),
            pltpu.VMEM((BATCH, D_IN), _f32),
            pltpu.VMEM((BATCH, D_IN), _f32),
            pltpu.VMEM((BATCH, D_IN), _f32),
            pltpu.VMEM_SHARED((N_PAD, D_IN), _f32),
            pltpu.VMEM_SHARED((N_PAD, D_IN), _f32),
        ] + [pltpu.SemaphoreType.DMA] * 10,
        compiler_params=pltpu.CompilerParams(use_tc_tiling_on_sc=False),
    )
    def sc_layer1(x_h, src_h, dst_h, w_h, z16_h, ones_h, agg_o, cnt_o,
                  sidx_b, didx_b, wv_b, r0b, r1b, r2b, onev,
                  aggacc, cntacc, i0, i1, i2, g0, g1, g2,
                  s0, s1, s2, semc):
        c = lax.axis_index("c")
        s = lax.axis_index("s")
        wid = c * 16 + s
        r0 = pl.multiple_of(s * RPT, 8)
        pltpu.sync_copy(z16_h.at[pl.ds(r0, RPT)], aggacc.at[pl.ds(r0, RPT)])
        pltpu.sync_copy(z16_h.at[pl.ds(r0, RPT)], cntacc.at[pl.ds(r0, RPT)])
        pltpu.sync_copy(ones_h, onev)
        plsc.subcore_barrier()

        nbase = NBAT // 32
        rem = NBAT % 32
        start = wid * nbase + jnp.minimum(wid, rem)
        nb = nbase + jnp.where(wid < rem, 1, 0)

        def cnt_update(t, k):
            @pl.when(t >= 1)
            def _():
                pltpu.make_async_copy(onev, cntacc.at[didx_b.at[k]],
                                      semc).wait()

            pltpu.async_copy(onev, cntacc.at[didx_b.at[k]], semc, add=True)

        _edge_pipeline(x_h, src_h, dst_h, w_h, start, nb,
                       sidx_b, didx_b, wv_b, (r0b, r1b, r2b),
                       (i0, i1, i2), (g0, g1, g2), (s0, s1, s2),
                       aggacc, D_IN, extra=cnt_update)
        pltpu.make_async_copy(onev, cntacc.at[didx_b.at[0]], semc).wait()
        plsc.subcore_barrier()
        pltpu.sync_copy(aggacc.at[pl.ds(r0, RPT)],
                        agg_o.at[c, pl.ds(r0, RPT)])
        pltpu.sync_copy(cntacc.at[pl.ds(r0, RPT)],
                        cnt_o.at[c, pl.ds(r0, RPT)])

    return sc_layer1


def _make_sc_agg_h():
    mesh = plsc.VectorSubcoreMesh(core_axis_name="c", subcore_axis_name="s")

    @functools.partial(
        pl.kernel,
        mesh=mesh,
        out_type=jax.ShapeDtypeStruct((4, N_PAD, 32), _f32),
        scratch_types=[
            pltpu.VMEM((3, BATCH), _i32),
            pltpu.VMEM((3, BATCH), _i32),
            pltpu.VMEM((3, BATCH), _f32),
            pltpu.VMEM((BATCH, D_IN), _i32),
            pltpu.VMEM((BATCH, D_IN), _i32),
            pltpu.VMEM((BATCH, D_IN), _i32),
            pltpu.VMEM((BATCH, 32), _f32),
            pltpu.VMEM((BATCH, 32), _f32),
            pltpu.VMEM((BATCH, 32), _f32),
            pltpu.VMEM_SHARED((N_PAD, 32), _f32),
        ] + [pltpu.SemaphoreType.DMA] * 9,
        compiler_params=pltpu.CompilerParams(use_tc_tiling_on_sc=False),
    )
    def sc_agg(h0_h, h1_h, h2_h, h3_h, src_h, dst_h, w_h, z32_h, out_o,
               sidx_b, didx_b, wv_b, r0b, r1b, r2b, f0b, f1b, f2b, acc,
               i0, i1, i2, g0, g1, g2, s0, s1, s2):
        c = lax.axis_index("c")
        s = lax.axis_index("s")
        r0 = pl.multiple_of(s * RPT, 8)

        nbase = NBAT // 16
        rem = NBAT % 16
        start = s * nbase + jnp.minimum(s, rem)
        nb = nbase + jnp.where(s < rem, 1, 0)

        def process(tab_h, chunk):
            pltpu.sync_copy(z32_h.at[pl.ds(r0, RPT)], acc.at[pl.ds(r0, RPT)])
            plsc.subcore_barrier()
            _edge_pipeline(tab_h, src_h, dst_h, w_h, start, nb,
                           sidx_b, didx_b, wv_b, (r0b, r1b, r2b),
                           (i0, i1, i2), (g0, g1, g2),
                           (s0, s1, s2), acc, 32,
                           stages=(f0b, f1b, f2b))
            plsc.subcore_barrier()
            pltpu.sync_copy(acc.at[pl.ds(r0, RPT)],
                            out_o.at[chunk, pl.ds(r0, RPT)])
            plsc.subcore_barrier()

        @pl.when(c == 0)
        def _():
            process(h0_h, 0)
            process(h1_h, 1)

        @pl.when(c == 1)
        def _():
            process(h2_h, 2)
            process(h3_h, 3)

    return sc_agg


_sc_layer1 = None
_sc_agg_h = None


def _get_sc_kernels():
    global _sc_layer1, _sc_agg_h
    if _sc_layer1 is None:
        _sc_layer1 = _make_sc_layer1()
        _sc_agg_h = _make_sc_agg_h()
    return _sc_layer1, _sc_agg_h

BLK = 2000
GRID = N // BLK


def _merge_rc(cnt_ref):
    cnt = cnt_ref[0] + cnt_ref[1]
    return 1.0 / jnp.maximum(cnt[:, 0:1], 1.0)


def _write_chunks(h, hout_refs):
    for k in range(4):
        lo = h[:, 32 * k:32 * k + 16]
        hi = h[:, 32 * k + 16:32 * k + 32]
        lo_b = lax.bitcast_convert_type(lo.astype(jnp.bfloat16), jnp.uint16)
        hi_b = lax.bitcast_convert_type(hi.astype(jnp.bfloat16), jnp.uint16)
        word = (lo_b.astype(jnp.uint32)
                | jnp.left_shift(hi_b.astype(jnp.uint32), 16))
        hout_refs[k][...] = lax.bitcast_convert_type(word, _i32)


def _read_chunks(refs):
    parts = []
    for r in refs:
        w = r[...]
        lo = lax.bitcast_convert_type(jnp.left_shift(w, 16), _f32)
        hi = lax.bitcast_convert_type(
            jnp.bitwise_and(w, jnp.int32(-65536)), _f32)
        parts.append(lo)
        parts.append(hi)
    return jnp.concatenate(parts, axis=1)


def _readout_update(i, h, gmp_ref, gsum_ref):
    bm = jnp.max(h, axis=0, keepdims=True)
    bs = jnp.sum(h, axis=0, keepdims=True)

    @pl.when(i == 0)
    def _():
        gmp_ref[...] = bm
        gsum_ref[...] = bs

    @pl.when(i > 0)
    def _():
        gmp_ref[...] = jnp.maximum(gmp_ref[...], bm)
        gsum_ref[...] = gsum_ref[...] + bs

    @pl.when(i == GRID - 1)
    def _():
        gsum_ref[...] = gsum_ref[...] * (1.0 / N)


def _tc_b1_body(aggp, cntp, xr, w1l, b1, w1r, h0, h1, h2, h3, gmp, gsum):
    i = pl.program_id(0)
    agg = (aggp[0] + aggp[1]) * _merge_rc(cntp)
    h = (jnp.dot(agg, w1l[...], preferred_element_type=_f32) + b1[...]
         + jnp.dot(xr[...], w1r[...], preferred_element_type=_f32))
    h = jnp.maximum(h, 0.0)
    _write_chunks(h, (h0, h1, h2, h3))
    _readout_update(i, h, gmp, gsum)


def _tc_b1(aggp, cntp, x, w1lT, b1, w1rT):
    return pl.pallas_call(
        _tc_b1_body,
        grid=(GRID,),
        in_specs=[
            pl.BlockSpec((2, BLK, D_IN), lambda i: (0, i, 0)),
            pl.BlockSpec((2, BLK, D_IN), lambda i: (0, i, 0)),
            pl.BlockSpec((BLK, D_IN), lambda i: (i, 0)),
            pl.BlockSpec((D_IN, H), lambda i: (0, 0)),
            pl.BlockSpec((1, H), lambda i: (0, 0)),
            pl.BlockSpec((D_IN, H), lambda i: (0, 0)),
        ],
        out_specs=[pl.BlockSpec((BLK, D_IN), lambda i: (i, 0))] * 4 + [
            pl.BlockSpec((1, H), lambda i: (0, 0)),
            pl.BlockSpec((1, H), lambda i: (0, 0)),
        ],
        out_shape=[jax.ShapeDtypeStruct((N, D_IN), _i32)] * 4 + [
            jax.ShapeDtypeStruct((1, H), _f32),
            jax.ShapeDtypeStruct((1, H), _f32),
        ],
    )(aggp, cntp, x, w1lT, b1, w1rT)


def _tc_mid_body(write_h, aggc, cntp, hp0, hp1, hp2, hp3, wl, b, wr, *outs):
    i = pl.program_id(0)
    if write_h:
        h0, h1, h2, h3, gmp, gsum = outs
    else:
        gmp, gsum = outs
    agg = jnp.concatenate([aggc[k] for k in range(4)], axis=1)
    agg = agg * _merge_rc(cntp)
    hp = _read_chunks((hp0, hp1, hp2, hp3))
    h = (jnp.dot(agg, wl[...], preferred_element_type=_f32) + b[...]
         + jnp.dot(hp, wr[...], preferred_element_type=_f32))
    h = jnp.maximum(h, 0.0)
    if write_h:
        _write_chunks(h, (h0, h1, h2, h3))
    _readout_update(i, h, gmp, gsum)


def _tc_mid(aggc, cntp, hprev, wlT, b, wrT, write_h):
    out_specs = [
        pl.BlockSpec((1, H), lambda i: (0, 0)),
        pl.BlockSpec((1, H), lambda i: (0, 0)),
    ]
    out_shape = [
        jax.ShapeDtypeStruct((1, H), _f32),
        jax.ShapeDtypeStruct((1, H), _f32),
    ]
    if write_h:
        out_specs = [pl.BlockSpec((BLK, D_IN), lambda i: (i, 0))] * 4 + out_specs
        out_shape = [jax.ShapeDtypeStruct((N, D_IN), _i32)] * 4 + out_shape
    return pl.pallas_call(
        functools.partial(_tc_mid_body, write_h),
        grid=(GRID,),
        in_specs=[
            pl.BlockSpec((4, BLK, 32), lambda i: (0, i, 0)),
            pl.BlockSpec((2, BLK, D_IN), lambda i: (0, i, 0)),
        ] + [pl.BlockSpec((BLK, D_IN), lambda i: (i, 0))] * 4 + [
            pl.BlockSpec((H, H), lambda i: (0, 0)),
            pl.BlockSpec((1, H), lambda i: (0, 0)),
            pl.BlockSpec((H, H), lambda i: (0, 0)),
        ],
        out_specs=out_specs,
        out_shape=out_shape,
    )(aggc, cntp, *hprev, wlT, b, wrT)


def _tc_head_body(g1, a1, g2, a2, g3, a3, w1, b1, w2, b2, w3, b3, gout):
    z = (jnp.concatenate([g1[...], a1[...]], axis=1)
         + jnp.concatenate([g2[...], a2[...]], axis=1)
         + jnp.concatenate([g3[...], a3[...]], axis=1))
    t = jnp.maximum(jnp.dot(z, w1[...], preferred_element_type=_f32) + b1[...], 0.0)
    t = jnp.maximum(jnp.dot(t, w2[...], preferred_element_type=_f32) + b2[...], 0.0)
    lg = jnp.dot(t, w3[...], preferred_element_type=_f32) + b3[...]
    sg = 1.0 / (1.0 + jnp.exp(-lg))
    col = lax.broadcasted_iota(_i32, (1, H), 1)
    lm = jnp.where(col < 2, sg, -jnp.inf)
    m = jnp.max(lm, axis=1, keepdims=True)
    e = jnp.where(col < 2, jnp.exp(lm - m), 0.0)
    gout[...] = lm - m - jnp.log(jnp.sum(e, axis=1, keepdims=True))


def _tc_head(g1, a1, g2, a2, g3, a3, w1T, b1, w2T, b2, w3T, b3):
    full = lambda shape: pl.BlockSpec(shape, lambda: tuple(0 for _ in shape))
    return pl.pallas_call(
        _tc_head_body,
        in_specs=[full((1, H))] * 6 + [
            full((2 * H, H)), full((1, H)), full((H, H)), full((1, H)),
            full((H, H)), full((1, H)),
        ],
        out_specs=full((1, H)),
        out_shape=jax.ShapeDtypeStruct((1, H), _f32),
    )(g1, a1, g2, a2, g3, a3, w1T, b1, w2T, b2, w3T, b3)


def kernel(x, edge_index, edge_weight, indices, W1l, b1l, W1r, W2l, b2l, W2r,
           W3l, b3l, W3r, Wlin1, blin1, Wlin2, blin2, Wlin3, blin3):
    src = edge_index[0]
    dst = edge_index[1]
    ew = edge_weight
    z16 = jnp.zeros((N_PAD, D_IN), _f32)
    z32 = jnp.zeros((N_PAD, 32), _f32)
    ones_rows = jnp.ones((BATCH, D_IN), _f32)

    w1lT = W1l.T
    w1rT = W1r.T
    w2lT = W2l.T
    w2rT = W2r.T
    w3lT = W3l.T
    w3rT = W3r.T
    wlin1T = Wlin1.T
    wlin2T = jnp.zeros((H, H), _f32).at[:, :64].set(Wlin2.T)
    b2pad = jnp.zeros((1, H), _f32).at[:, :64].set(blin2[None, :])
    wlin3T = jnp.zeros((H, H), _f32).at[:64, :2].set(Wlin3.T)
    b3row = jnp.zeros((1, H), _f32).at[:, :2].set(blin3[None, :])

    sc_layer1, sc_agg_h = _get_sc_kernels()
    aggp, cntp = sc_layer1(x, src, dst, ew, z16, ones_rows)
    h10, h11, h12, h13, g1, a1 = _tc_b1(aggp, cntp, x, w1lT, b1l[None, :],
                                        w1rT)
    agg2 = sc_agg_h(h10, h11, h12, h13, src, dst, ew, z32)
    h20, h21, h22, h23, g2, a2 = _tc_mid(agg2, cntp, (h10, h11, h12, h13),
                                         w2lT, b2l[None, :], w2rT, True)
    agg3 = sc_agg_h(h20, h21, h22, h23, src, dst, ew, z32)
    g3, a3 = _tc_mid(agg3, cntp, (h20, h21, h22, h23), w3lT, b3l[None, :],
                     w3rT, False)

    out = _tc_head(g1, a1, g2, a2, g3, a3, wlin1T, blin1[None, :],
                   wlin2T, b2pad, wlin3T, b3row)
    sel = jnp.take(out[:, :2], indices, axis=0)
    return sel


# speedup vs baseline: 1.7290x; 1.7290x over previous
import functools

import jax
import jax.numpy as jnp
from jax import lax
from jax.experimental import pallas as pl
from jax.experimental.pallas import tpu as pltpu
from jax.experimental.pallas import tpu_sc as plsc

N = 50000
E = 800000
D_IN = 16
H = 128
BATCH = 256
NBAT = E // BATCH
RPT = 3128
N_PAD = 16 * RPT

_f32 = jnp.float32
_i32 = jnp.int32


def _scale_rows(wv_b, k, rows, width):
    nv = width // 16

    def group_body(g, carry):
        wvec = wv_b[k, pl.ds(pl.multiple_of(g * 16, 16), 16)]
        for kk in range(16):
            j = g * 16 + kk
            spl = jnp.full((16,), wvec[kk], dtype=_f32)
            for v in range(nv):
                rows[j, pl.ds(16 * v, 16)] = rows[j, pl.ds(16 * v, 16)] * spl
        return carry

    lax.fori_loop(0, BATCH // 16, group_body, 0)


def _edge_pipeline(tab_h, src_h, dst_h, w_h, start, pn, sidx_b, didx_b,
                   wv_b, bufs, semi, semg, sems, acc, width, extra=None):
    def idx_load(t, k):
        eb = pl.multiple_of((start + t) * BATCH, BATCH)
        pltpu.async_copy(src_h.at[pl.ds(eb, BATCH)], sidx_b.at[k], semi[k])
        pltpu.async_copy(dst_h.at[pl.ds(eb, BATCH)], didx_b.at[k], semi[k])
        pltpu.async_copy(w_h.at[pl.ds(eb, BATCH)], wv_b.at[k], semi[k])

    def wait_idx(k):
        pltpu.make_async_copy(src_h.at[pl.ds(0, BATCH)], sidx_b.at[k],
                              semi[k]).wait()
        pltpu.make_async_copy(dst_h.at[pl.ds(0, BATCH)], didx_b.at[k],
                              semi[k]).wait()
        pltpu.make_async_copy(w_h.at[pl.ds(0, BATCH)], wv_b.at[k],
                              semi[k]).wait()

    def gather(k):
        pltpu.async_copy(tab_h.at[sidx_b.at[k]], bufs[k], semg[k])

    def wait_gather(k):
        pltpu.make_async_copy(tab_h.at[sidx_b.at[k]], bufs[k], semg[k]).wait()

    def wait_scatter(k):
        pltpu.make_async_copy(bufs[k], acc.at[didx_b.at[k]], sems[k]).wait()

    idx_load(0, 0)
    idx_load(1, 1)
    wait_idx(0)
    gather(0)

    def batch_body(t, carry):
        for k in range(3):
            @pl.when(lax.rem(t, 3) == k)
            def _(k=k):
                k1 = (k + 1) % 3
                k2 = (k + 2) % 3

                @pl.when(t + 1 < pn)
                def _():
                    wait_idx(k1)
                    gather(k1)

                wait_gather(k)
                _scale_rows(wv_b, k, bufs[k], width)
                pltpu.async_copy(bufs[k], acc.at[didx_b.at[k]], sems[k],
                                 add=True)
                if extra is not None:
                    extra(t, k)

                @pl.when(t >= 1)
                def _():
                    wait_scatter(k2)

                @pl.when(t + 2 < pn)
                def _():
                    idx_load(t + 2, k2)
        return carry

    lax.fori_loop(0, pn, batch_body, 0)
    td = pn - 1
    for k in range(3):
        @pl.when(lax.rem(td, 3) == k)
        def _(k=k):
            wait_scatter(k)


def _make_sc_layer1():
    mesh = plsc.VectorSubcoreMesh(core_axis_name="c", subcore_axis_name="s")

    @functools.partial(
        pl.kernel,
        mesh=mesh,
        out_type=[
            jax.ShapeDtypeStruct((2, N_PAD, D_IN), _f32),
            jax.ShapeDtypeStruct((2, N_PAD, D_IN), _f32),
        ],
        scratch_types=[
            pltpu.VMEM((3, BATCH), _i32),
            pltpu.VMEM((3, BATCH), _i32),
            pltpu.VMEM((3, BATCH), _f32),
            pltpu.VMEM((BATCH, D_IN), _f32),
            pltpu.VMEM((BATCH, D_IN), _f32),
            pltpu.VMEM((BATCH, D_IN), _f32),
            pltpu.VMEM((BATCH, D_IN), _f32),
            pltpu.VMEM_SHARED((N_PAD, D_IN), _f32),
            pltpu.VMEM_SHARED((N_PAD, D_IN), _f32),
        ] + [pltpu.SemaphoreType.DMA] * 10,
        compiler_params=pltpu.CompilerParams(use_tc_tiling_on_sc=False),
    )
    def sc_layer1(x_h, src_h, dst_h, w_h, z16_h, ones_h, agg_o, cnt_o,
                  sidx_b, didx_b, wv_b, r0b, r1b, r2b, onev,
                  aggacc, cntacc, i0, i1, i2, g0, g1, g2,
                  s0, s1, s2, semc):
        c = lax.axis_index("c")
        s = lax.axis_index("s")
        wid = c * 16 + s
        r0 = pl.multiple_of(s * RPT, 8)
        pltpu.sync_copy(z16_h.at[pl.ds(r0, RPT)], aggacc.at[pl.ds(r0, RPT)])
        pltpu.sync_copy(z16_h.at[pl.ds(r0, RPT)], cntacc.at[pl.ds(r0, RPT)])
        pltpu.sync_copy(ones_h, onev)
        plsc.subcore_barrier()

        nbase = NBAT // 32
        rem = NBAT % 32
        start = wid * nbase + jnp.minimum(wid, rem)
        nb = nbase + jnp.where(wid < rem, 1, 0)

        def cnt_update(t, k):
            @pl.when(t >= 1)
            def _():
                pltpu.make_async_copy(onev, cntacc.at[didx_b.at[k]],
                                      semc).wait()

            pltpu.async_copy(onev, cntacc.at[didx_b.at[k]], semc, add=True)

        _edge_pipeline(x_h, src_h, dst_h, w_h, start, nb,
                       sidx_b, didx_b, wv_b, (r0b, r1b, r2b),
                       (i0, i1, i2), (g0, g1, g2), (s0, s1, s2),
                       aggacc, D_IN, extra=cnt_update)
        pltpu.make_async_copy(onev, cntacc.at[didx_b.at[0]], semc).wait()
        plsc.subcore_barrier()
        pltpu.sync_copy(aggacc.at[pl.ds(r0, RPT)],
                        agg_o.at[c, pl.ds(r0, RPT)])
        pltpu.sync_copy(cntacc.at[pl.ds(r0, RPT)],
                        cnt_o.at[c, pl.ds(r0, RPT)])

    return sc_layer1


def _make_sc_agg_h():
    mesh = plsc.VectorSubcoreMesh(core_axis_name="c", subcore_axis_name="s")

    @functools.partial(
        pl.kernel,
        mesh=mesh,
        out_type=jax.ShapeDtypeStruct((4, N_PAD, 32), _f32),
        scratch_types=[
            pltpu.VMEM((3, BATCH), _i32),
            pltpu.VMEM((3, BATCH), _i32),
            pltpu.VMEM((3, BATCH), _f32),
            pltpu.VMEM((BATCH, 32), _f32),
            pltpu.VMEM((BATCH, 32), _f32),
            pltpu.VMEM((BATCH, 32), _f32),
            pltpu.VMEM_SHARED((N_PAD, 32), _f32),
        ] + [pltpu.SemaphoreType.DMA] * 9,
        compiler_params=pltpu.CompilerParams(use_tc_tiling_on_sc=False),
    )
    def sc_agg(h0_h, h1_h, h2_h, h3_h, src_h, dst_h, w_h, z32_h, out_o,
               sidx_b, didx_b, wv_b, r0b, r1b, r2b, acc,
               i0, i1, i2, g0, g1, g2, s0, s1, s2):
        c = lax.axis_index("c")
        s = lax.axis_index("s")
        r0 = pl.multiple_of(s * RPT, 8)

        nbase = NBAT // 16
        rem = NBAT % 16
        start = s * nbase + jnp.minimum(s, rem)
        nb = nbase + jnp.where(s < rem, 1, 0)

        def process(tab_h, chunk):
            pltpu.sync_copy(z32_h.at[pl.ds(r0, RPT)], acc.at[pl.ds(r0, RPT)])
            plsc.subcore_barrier()
            _edge_pipeline(tab_h, src_h, dst_h, w_h, start, nb,
                           sidx_b, didx_b, wv_b, (r0b, r1b, r2b),
                           (i0, i1, i2), (g0, g1, g2),
                           (s0, s1, s2), acc, 32)
            plsc.subcore_barrier()
            pltpu.sync_copy(acc.at[pl.ds(r0, RPT)],
                            out_o.at[chunk, pl.ds(r0, RPT)])
            plsc.subcore_barrier()

        @pl.when(c == 0)
        def _():
            process(h0_h, 0)
            process(h1_h, 1)

        @pl.when(c == 1)
        def _():
            process(h2_h, 2)
            process(h3_h, 3)

    return sc_agg


_sc_layer1 = None
_sc_agg_h = None


def _get_sc_kernels():
    global _sc_layer1, _sc_agg_h
    if _sc_layer1 is None:
        _sc_layer1 = _make_sc_layer1()
        _sc_agg_h = _make_sc_agg_h()
    return _sc_layer1, _sc_agg_h

BLK = 2000
GRID = N // BLK


def _merge_rc(cnt_ref):
    cnt = cnt_ref[0] + cnt_ref[1]
    return 1.0 / jnp.maximum(cnt[:, 0:1], 1.0)


def _write_chunks(h, hout_refs):
    for k in range(4):
        hout_refs[k][...] = h[:, 32 * k:32 * k + 32]


def _readout_update(i, h, gmp_ref, gsum_ref):
    bm = jnp.max(h, axis=0, keepdims=True)
    bs = jnp.sum(h, axis=0, keepdims=True)

    @pl.when(i == 0)
    def _():
        gmp_ref[...] = bm
        gsum_ref[...] = bs

    @pl.when(i > 0)
    def _():
        gmp_ref[...] = jnp.maximum(gmp_ref[...], bm)
        gsum_ref[...] = gsum_ref[...] + bs

    @pl.when(i == GRID - 1)
    def _():
        gsum_ref[...] = gsum_ref[...] * (1.0 / N)


def _tc_self_body(hp0, hp1, hp2, hp3, wr, sout):
    hp = jnp.concatenate([hp0[...], hp1[...], hp2[...], hp3[...]], axis=1)
    sout[...] = jnp.dot(hp, wr[...], preferred_element_type=_f32)


def _tc_self(hprev, wrT):
    return pl.pallas_call(
        _tc_self_body,
        grid=(GRID,),
        in_specs=[pl.BlockSpec((BLK, 32), lambda i: (i, 0))] * 4 + [
            pl.BlockSpec((H, H), lambda i: (0, 0)),
        ],
        out_specs=pl.BlockSpec((BLK, H), lambda i: (i, 0)),
        out_shape=jax.ShapeDtypeStruct((N, H), _f32),
    )(*hprev, wrT)


def _tc_self_x_body(xr, wr, sout):
    sout[...] = jnp.dot(xr[...], wr[...], preferred_element_type=_f32)


def _tc_self_x(x, wrT):
    return pl.pallas_call(
        _tc_self_x_body,
        grid=(GRID,),
        in_specs=[
            pl.BlockSpec((BLK, D_IN), lambda i: (i, 0)),
            pl.BlockSpec((D_IN, H), lambda i: (0, 0)),
        ],
        out_specs=pl.BlockSpec((BLK, H), lambda i: (i, 0)),
        out_shape=jax.ShapeDtypeStruct((N, H), _f32),
    )(x, wrT)


def _tc_b1_body(aggp, cntp, sterm, w1l, b1, h0, h1, h2, h3, gmp, gsum):
    i = pl.program_id(0)
    agg = (aggp[0] + aggp[1]) * _merge_rc(cntp)
    h = (jnp.dot(agg, w1l[...], preferred_element_type=_f32) + b1[...]
         + sterm[...])
    h = jnp.maximum(h, 0.0)
    _write_chunks(h, (h0, h1, h2, h3))
    _readout_update(i, h, gmp, gsum)


def _tc_b1(aggp, cntp, sterm, w1lT, b1):
    return pl.pallas_call(
        _tc_b1_body,
        grid=(GRID,),
        in_specs=[
            pl.BlockSpec((2, BLK, D_IN), lambda i: (0, i, 0)),
            pl.BlockSpec((2, BLK, D_IN), lambda i: (0, i, 0)),
            pl.BlockSpec((BLK, H), lambda i: (i, 0)),
            pl.BlockSpec((D_IN, H), lambda i: (0, 0)),
            pl.BlockSpec((1, H), lambda i: (0, 0)),
        ],
        out_specs=[pl.BlockSpec((BLK, 32), lambda i: (i, 0))] * 4 + [
            pl.BlockSpec((1, H), lambda i: (0, 0)),
            pl.BlockSpec((1, H), lambda i: (0, 0)),
        ],
        out_shape=[jax.ShapeDtypeStruct((N, 32), _f32)] * 4 + [
            jax.ShapeDtypeStruct((1, H), _f32),
            jax.ShapeDtypeStruct((1, H), _f32),
        ],
    )(aggp, cntp, sterm, w1lT, b1)


def _tc_mid_body(write_h, aggc, cntp, sterm, wl, b, *outs):
    i = pl.program_id(0)
    if write_h:
        h0, h1, h2, h3, gmp, gsum = outs
    else:
        gmp, gsum = outs
    agg = jnp.concatenate([aggc[k] for k in range(4)], axis=1)
    agg = agg * _merge_rc(cntp)
    h = (jnp.dot(agg, wl[...], preferred_element_type=_f32) + b[...]
         + sterm[...])
    h = jnp.maximum(h, 0.0)
    if write_h:
        _write_chunks(h, (h0, h1, h2, h3))
    _readout_update(i, h, gmp, gsum)


def _tc_mid(aggc, cntp, sterm, wlT, b, write_h):
    out_specs = [
        pl.BlockSpec((1, H), lambda i: (0, 0)),
        pl.BlockSpec((1, H), lambda i: (0, 0)),
    ]
    out_shape = [
        jax.ShapeDtypeStruct((1, H), _f32),
        jax.ShapeDtypeStruct((1, H), _f32),
    ]
    if write_h:
        out_specs = [pl.BlockSpec((BLK, 32), lambda i: (i, 0))] * 4 + out_specs
        out_shape = [jax.ShapeDtypeStruct((N, 32), _f32)] * 4 + out_shape
    return pl.pallas_call(
        functools.partial(_tc_mid_body, write_h),
        grid=(GRID,),
        in_specs=[
            pl.BlockSpec((4, BLK, 32), lambda i: (0, i, 0)),
            pl.BlockSpec((2, BLK, D_IN), lambda i: (0, i, 0)),
            pl.BlockSpec((BLK, H), lambda i: (i, 0)),
            pl.BlockSpec((H, H), lambda i: (0, 0)),
            pl.BlockSpec((1, H), lambda i: (0, 0)),
        ],
        out_specs=out_specs,
        out_shape=out_shape,
    )(aggc, cntp, sterm, wlT, b)


def _tc_head_body(g1, a1, g2, a2, g3, a3, w1, b1, w2, b2, w3, b3, gout):
    z = (jnp.concatenate([g1[...], a1[...]], axis=1)
         + jnp.concatenate([g2[...], a2[...]], axis=1)
         + jnp.concatenate([g3[...], a3[...]], axis=1))
    t = jnp.maximum(jnp.dot(z, w1[...], preferred_element_type=_f32) + b1[...], 0.0)
    t = jnp.maximum(jnp.dot(t, w2[...], preferred_element_type=_f32) + b2[...], 0.0)
    lg = jnp.dot(t, w3[...], preferred_element_type=_f32) + b3[...]
    sg = 1.0 / (1.0 + jnp.exp(-lg))
    col = lax.broadcasted_iota(_i32, (1, H), 1)
    lm = jnp.where(col < 2, sg, -jnp.inf)
    m = jnp.max(lm, axis=1, keepdims=True)
    e = jnp.where(col < 2, jnp.exp(lm - m), 0.0)
    gout[...] = lm - m - jnp.log(jnp.sum(e, axis=1, keepdims=True))


def _tc_head(g1, a1, g2, a2, g3, a3, w1T, b1, w2T, b2, w3T, b3):
    full = lambda shape: pl.BlockSpec(shape, lambda: tuple(0 for _ in shape))
    return pl.pallas_call(
        _tc_head_body,
        in_specs=[full((1, H))] * 6 + [
            full((2 * H, H)), full((1, H)), full((H, H)), full((1, H)),
            full((H, H)), full((1, H)),
        ],
        out_specs=full((1, H)),
        out_shape=jax.ShapeDtypeStruct((1, H), _f32),
    )(g1, a1, g2, a2, g3, a3, w1T, b1, w2T, b2, w3T, b3)


def kernel(x, edge_index, edge_weight, indices, W1l, b1l, W1r, W2l, b2l, W2r,
           W3l, b3l, W3r, Wlin1, blin1, Wlin2, blin2, Wlin3, blin3):
    src = edge_index[0]
    dst = edge_index[1]
    ew = edge_weight
    z16 = jnp.zeros((N_PAD, D_IN), _f32)
    z32 = jnp.zeros((N_PAD, 32), _f32)
    ones_rows = jnp.ones((BATCH, D_IN), _f32)

    w1lT = W1l.T
    w1rT = W1r.T
    w2lT = W2l.T
    w2rT = W2r.T
    w3lT = W3l.T
    w3rT = W3r.T
    wlin1T = Wlin1.T
    wlin2T = jnp.zeros((H, H), _f32).at[:, :64].set(Wlin2.T)
    b2pad = jnp.zeros((1, H), _f32).at[:, :64].set(blin2[None, :])
    wlin3T = jnp.zeros((H, H), _f32).at[:64, :2].set(Wlin3.T)
    b3row = jnp.zeros((1, H), _f32).at[:, :2].set(blin3[None, :])

    sc_layer1, sc_agg_h = _get_sc_kernels()
    aggp, cntp = sc_layer1(x, src, dst, ew, z16, ones_rows)
    s1 = _tc_self_x(x, w1rT)
    h10, h11, h12, h13, g1, a1 = _tc_b1(aggp, cntp, s1, w1lT, b1l[None, :])
    agg2 = sc_agg_h(h10, h11, h12, h13, src, dst, ew, z32)
    s2 = _tc_self((h10, h11, h12, h13), w2rT)
    h20, h21, h22, h23, g2, a2 = _tc_mid(agg2, cntp, s2, w2lT, b2l[None, :],
                                         True)
    agg3 = sc_agg_h(h20, h21, h22, h23, src, dst, ew, z32)
    s3 = _tc_self((h20, h21, h22, h23), w3rT)
    g3, a3 = _tc_mid(agg3, cntp, s3, w3lT, b3l[None, :], False)

    out = _tc_head(g1, a1, g2, a2, g3, a3, wlin1T, blin1[None, :],
                   wlin2T, b2pad, wlin3T, b3row)
    sel = jnp.take(out[:, :2], indices, axis=0)
    return sel


# speedup vs baseline: 1.7329x; 1.0022x over previous
import functools

import jax
import jax.numpy as jnp
from jax import lax
from jax.experimental import pallas as pl
from jax.experimental.pallas import tpu as pltpu
from jax.experimental.pallas import tpu_sc as plsc

N = 50000
E = 800000
D_IN = 16
H = 128
BATCH = 256
NBAT = E // BATCH
RPT = 3128
N_PAD = 16 * RPT

_f32 = jnp.float32
_i32 = jnp.int32


def _scale_rows(wv_b, k, rows, width):
    nv = width // 16

    def group_body(g, carry):
        wvec = wv_b[k, pl.ds(pl.multiple_of(g * 16, 16), 16)]
        for kk in range(16):
            j = g * 16 + kk
            spl = jnp.full((16,), wvec[kk], dtype=_f32)
            for v in range(nv):
                rows[j, pl.ds(16 * v, 16)] = rows[j, pl.ds(16 * v, 16)] * spl
        return carry

    lax.fori_loop(0, BATCH // 16, group_body, 0)


def _edge_pipeline(tab_h, src_h, dst_h, w_h, start, pn, sidx_b, didx_b,
                   wv_b, bufs, semi, semg, sems, acc, width, extra=None):
    def idx_load(t, k):
        eb = pl.multiple_of((start + t) * BATCH, BATCH)
        pltpu.async_copy(src_h.at[pl.ds(eb, BATCH)], sidx_b.at[k], semi[k])
        pltpu.async_copy(dst_h.at[pl.ds(eb, BATCH)], didx_b.at[k], semi[k])
        pltpu.async_copy(w_h.at[pl.ds(eb, BATCH)], wv_b.at[k], semi[k])

    def wait_idx(k):
        pltpu.make_async_copy(src_h.at[pl.ds(0, BATCH)], sidx_b.at[k],
                              semi[k]).wait()
        pltpu.make_async_copy(dst_h.at[pl.ds(0, BATCH)], didx_b.at[k],
                              semi[k]).wait()
        pltpu.make_async_copy(w_h.at[pl.ds(0, BATCH)], wv_b.at[k],
                              semi[k]).wait()

    def gather(k):
        pltpu.async_copy(tab_h.at[sidx_b.at[k]], bufs[k], semg[k])

    def wait_gather(k):
        pltpu.make_async_copy(tab_h.at[sidx_b.at[k]], bufs[k], semg[k]).wait()

    def wait_scatter(k):
        pltpu.make_async_copy(bufs[k], acc.at[didx_b.at[k]], sems[k]).wait()

    idx_load(0, 0)
    idx_load(1, 1)
    wait_idx(0)
    gather(0)

    def batch_body(t, carry):
        for k in range(3):
            @pl.when(lax.rem(t, 3) == k)
            def _(k=k):
                k1 = (k + 1) % 3
                k2 = (k + 2) % 3

                @pl.when(t + 1 < pn)
                def _():
                    wait_idx(k1)
                    gather(k1)

                wait_gather(k)
                _scale_rows(wv_b, k, bufs[k], width)
                pltpu.async_copy(bufs[k], acc.at[didx_b.at[k]], sems[k],
                                 add=True)
                if extra is not None:
                    extra(t, k)

                @pl.when(t >= 1)
                def _():
                    wait_scatter(k2)

                @pl.when(t + 2 < pn)
                def _():
                    idx_load(t + 2, k2)
        return carry

    lax.fori_loop(0, pn, batch_body, 0)
    td = pn - 1
    for k in range(3):
        @pl.when(lax.rem(td, 3) == k)
        def _(k=k):
            wait_scatter(k)


def _make_sc_layer1():
    mesh = plsc.VectorSubcoreMesh(core_axis_name="c", subcore_axis_name="s")

    @functools.partial(
        pl.kernel,
        mesh=mesh,
        out_type=[
            jax.ShapeDtypeStruct((2, N_PAD, D_IN), _f32),
            jax.ShapeDtypeStruct((2, N_PAD, D_IN), _f32),
        ],
        scratch_types=[
            pltpu.VMEM((3, BATCH), _i32),
            pltpu.VMEM((3, BATCH), _i32),
            pltpu.VMEM((3, BATCH), _f32),
            pltpu.VMEM((BATCH, D_IN), _f32),
            pltpu.VMEM((BATCH, D_IN), _f32),
            pltpu.VMEM((BATCH, D_IN), _f32),
            pltpu.VMEM((BATCH, D_IN), _f32),
            pltpu.VMEM_SHARED((N_PAD, D_IN), _f32),
            pltpu.VMEM_SHARED((N_PAD, D_IN), _f32),
        ] + [pltpu.SemaphoreType.DMA] * 10,
        compiler_params=pltpu.CompilerParams(use_tc_tiling_on_sc=False),
    )
    def sc_layer1(x_h, src_h, dst_h, w_h, z16_h, ones_h, agg_o, cnt_o,
                  sidx_b, didx_b, wv_b, r0b, r1b, r2b, onev,
                  aggacc, cntacc, i0, i1, i2, g0, g1, g2,
                  s0, s1, s2, semc):
        c = lax.axis_index("c")
        s = lax.axis_index("s")
        wid = c * 16 + s
        r0 = pl.multiple_of(s * RPT, 8)
        pltpu.sync_copy(z16_h.at[pl.ds(r0, RPT)], aggacc.at[pl.ds(r0, RPT)])
        pltpu.sync_copy(z16_h.at[pl.ds(r0, RPT)], cntacc.at[pl.ds(r0, RPT)])
        pltpu.sync_copy(ones_h, onev)
        plsc.subcore_barrier()

        nbase = NBAT // 32
        rem = NBAT % 32
        start = wid * nbase + jnp.minimum(wid, rem)
        nb = nbase + jnp.where(wid < rem, 1, 0)

        def cnt_update(t, k):
            @pl.when(t >= 1)
            def _():
                pltpu.make_async_copy(onev, cntacc.at[didx_b.at[k]],
                                      semc).wait()

            pltpu.async_copy(onev, cntacc.at[didx_b.at[k]], semc, add=True)

        _edge_pipeline(x_h, src_h, dst_h, w_h, start, nb,
                       sidx_b, didx_b, wv_b, (r0b, r1b, r2b),
                       (i0, i1, i2), (g0, g1, g2), (s0, s1, s2),
                       aggacc, D_IN, extra=cnt_update)
        pltpu.make_async_copy(onev, cntacc.at[didx_b.at[0]], semc).wait()
        plsc.subcore_barrier()
        pltpu.sync_copy(aggacc.at[pl.ds(r0, RPT)],
                        agg_o.at[c, pl.ds(r0, RPT)])
        pltpu.sync_copy(cntacc.at[pl.ds(r0, RPT)],
                        cnt_o.at[c, pl.ds(r0, RPT)])

    return sc_layer1


def _make_sc_agg_h():
    mesh = plsc.VectorSubcoreMesh(core_axis_name="c", subcore_axis_name="s")

    @functools.partial(
        pl.kernel,
        mesh=mesh,
        out_type=jax.ShapeDtypeStruct((4, N_PAD, 32), _f32),
        scratch_types=[
            pltpu.VMEM((3, BATCH), _i32),
            pltpu.VMEM((3, BATCH), _i32),
            pltpu.VMEM((3, BATCH), _f32),
            pltpu.VMEM((BATCH, 32), _f32),
            pltpu.VMEM((BATCH, 32), _f32),
            pltpu.VMEM((BATCH, 32), _f32),
            pltpu.VMEM_SHARED((N_PAD, 32), _f32),
        ] + [pltpu.SemaphoreType.DMA] * 9,
        compiler_params=pltpu.CompilerParams(use_tc_tiling_on_sc=False),
    )
    def sc_agg(h0_h, h1_h, h2_h, h3_h, src_h, dst_h, w_h, z32_h, out_o,
               sidx_b, didx_b, wv_b, r0b, r1b, r2b, acc,
               i0, i1, i2, g0, g1, g2, s0, s1, s2):
        c = lax.axis_index("c")
        s = lax.axis_index("s")
        r0 = pl.multiple_of(s * RPT, 8)

        nbase = NBAT // 16
        rem = NBAT % 16
        start = s * nbase + jnp.minimum(s, rem)
        nb = nbase + jnp.where(s < rem, 1, 0)

        def process(tab_h, chunk):
            pltpu.sync_copy(z32_h.at[pl.ds(r0, RPT)], acc.at[pl.ds(r0, RPT)])
            plsc.subcore_barrier()
            _edge_pipeline(tab_h, src_h, dst_h, w_h, start, nb,
                           sidx_b, didx_b, wv_b, (r0b, r1b, r2b),
                           (i0, i1, i2), (g0, g1, g2),
                           (s0, s1, s2), acc, 32)
            plsc.subcore_barrier()
            pltpu.sync_copy(acc.at[pl.ds(r0, RPT)],
                            out_o.at[chunk, pl.ds(r0, RPT)])
            plsc.subcore_barrier()

        @pl.when(c == 0)
        def _():
            process(h0_h, 0)
            process(h1_h, 1)

        @pl.when(c == 1)
        def _():
            process(h2_h, 2)
            process(h3_h, 3)

    return sc_agg


_sc_layer1 = None
_sc_agg_h = None


def _get_sc_kernels():
    global _sc_layer1, _sc_agg_h
    if _sc_layer1 is None:
        _sc_layer1 = _make_sc_layer1()
        _sc_agg_h = _make_sc_agg_h()
    return _sc_layer1, _sc_agg_h

BLK = 2000
GRID = N // BLK


def _merge_rc(cnt_ref):
    cnt = cnt_ref[0] + cnt_ref[1]
    return 1.0 / jnp.maximum(cnt[:, 0:1], 1.0)


def _write_chunks(h, hout_refs):
    for k in range(4):
        hout_refs[k][...] = h[:, 32 * k:32 * k + 32]


def _readout_update(i, h, gmp_ref, gsum_ref):
    bm = jnp.max(h, axis=0, keepdims=True)
    bs = jnp.sum(h, axis=0, keepdims=True)

    @pl.when(i == 0)
    def _():
        gmp_ref[...] = bm
        gsum_ref[...] = bs

    @pl.when(i > 0)
    def _():
        gmp_ref[...] = jnp.maximum(gmp_ref[...], bm)
        gsum_ref[...] = gsum_ref[...] + bs

    @pl.when(i == GRID - 1)
    def _():
        gsum_ref[...] = gsum_ref[...] * (1.0 / N)


def _tc_b1_body(aggp, cntp, xr, w1l, b1, w1r, h0, h1, h2, h3, gmp, gsum):
    i = pl.program_id(0)
    agg = (aggp[0] + aggp[1]) * _merge_rc(cntp)
    h = (jnp.dot(agg, w1l[...], preferred_element_type=_f32) + b1[...]
         + jnp.dot(xr[...], w1r[...], preferred_element_type=_f32))
    h = jnp.maximum(h, 0.0)
    _write_chunks(h, (h0, h1, h2, h3))
    _readout_update(i, h, gmp, gsum)


def _tc_b1(aggp, cntp, x, w1lT, b1, w1rT):
    return pl.pallas_call(
        _tc_b1_body,
        grid=(GRID,),
        in_specs=[
            pl.BlockSpec((2, BLK, D_IN), lambda i: (0, i, 0)),
            pl.BlockSpec((2, BLK, D_IN), lambda i: (0, i, 0)),
            pl.BlockSpec((BLK, D_IN), lambda i: (i, 0)),
            pl.BlockSpec((D_IN, H), lambda i: (0, 0)),
            pl.BlockSpec((1, H), lambda i: (0, 0)),
            pl.BlockSpec((D_IN, H), lambda i: (0, 0)),
        ],
        out_specs=[pl.BlockSpec((BLK, 32), lambda i: (i, 0))] * 4 + [
            pl.BlockSpec((1, H), lambda i: (0, 0)),
            pl.BlockSpec((1, H), lambda i: (0, 0)),
        ],
        out_shape=[jax.ShapeDtypeStruct((N, 32), _f32)] * 4 + [
            jax.ShapeDtypeStruct((1, H), _f32),
            jax.ShapeDtypeStruct((1, H), _f32),
        ],
    )(aggp, cntp, x, w1lT, b1, w1rT)


def _tc_mid_body(write_h, aggc, cntp, hp0, hp1, hp2, hp3, wl, b, wr, *outs):
    i = pl.program_id(0)
    if write_h:
        h0, h1, h2, h3, gmp, gsum = outs
    else:
        gmp, gsum = outs
    agg = jnp.concatenate([aggc[k] for k in range(4)], axis=1)
    agg = agg * _merge_rc(cntp)
    hp = jnp.concatenate([hp0[...], hp1[...], hp2[...], hp3[...]], axis=1)
    h = (jnp.dot(agg, wl[...], preferred_element_type=_f32) + b[...]
         + jnp.dot(hp, wr[...], preferred_element_type=_f32))
    h = jnp.maximum(h, 0.0)
    if write_h:
        _write_chunks(h, (h0, h1, h2, h3))
    _readout_update(i, h, gmp, gsum)


def _tc_mid(aggc, cntp, hprev, wlT, b, wrT, write_h):
    out_specs = [
        pl.BlockSpec((1, H), lambda i: (0, 0)),
        pl.BlockSpec((1, H), lambda i: (0, 0)),
    ]
    out_shape = [
        jax.ShapeDtypeStruct((1, H), _f32),
        jax.ShapeDtypeStruct((1, H), _f32),
    ]
    if write_h:
        out_specs = [pl.BlockSpec((BLK, 32), lambda i: (i, 0))] * 4 + out_specs
        out_shape = [jax.ShapeDtypeStruct((N, 32), _f32)] * 4 + out_shape
    return pl.pallas_call(
        functools.partial(_tc_mid_body, write_h),
        grid=(GRID,),
        in_specs=[
            pl.BlockSpec((4, BLK, 32), lambda i: (0, i, 0)),
            pl.BlockSpec((2, BLK, D_IN), lambda i: (0, i, 0)),
        ] + [pl.BlockSpec((BLK, 32), lambda i: (i, 0))] * 4 + [
            pl.BlockSpec((H, H), lambda i: (0, 0)),
            pl.BlockSpec((1, H), lambda i: (0, 0)),
            pl.BlockSpec((H, H), lambda i: (0, 0)),
        ],
        out_specs=out_specs,
        out_shape=out_shape,
    )(aggc, cntp, *hprev, wlT, b, wrT)


def _tc_head_body(g1, a1, g2, a2, g3, a3, w1, b1, w2, b2, w3, b3, gout):
    z = (jnp.concatenate([g1[...], a1[...]], axis=1)
         + jnp.concatenate([g2[...], a2[...]], axis=1)
         + jnp.concatenate([g3[...], a3[...]], axis=1))
    t = jnp.maximum(jnp.dot(z, w1[...], preferred_element_type=_f32) + b1[...], 0.0)
    t = jnp.maximum(jnp.dot(t, w2[...], preferred_element_type=_f32) + b2[...], 0.0)
    lg = jnp.dot(t, w3[...], preferred_element_type=_f32) + b3[...]
    sg = 1.0 / (1.0 + jnp.exp(-lg))
    col = lax.broadcasted_iota(_i32, (1, H), 1)
    lm = jnp.where(col < 2, sg, -jnp.inf)
    m = jnp.max(lm, axis=1, keepdims=True)
    e = jnp.where(col < 2, jnp.exp(lm - m), 0.0)
    gout[...] = lm - m - jnp.log(jnp.sum(e, axis=1, keepdims=True))


def _tc_head(g1, a1, g2, a2, g3, a3, w1T, b1, w2T, b2, w3T, b3):
    full = lambda shape: pl.BlockSpec(shape, lambda: tuple(0 for _ in shape))
    return pl.pallas_call(
        _tc_head_body,
        in_specs=[full((1, H))] * 6 + [
            full((2 * H, H)), full((1, H)), full((H, H)), full((1, H)),
            full((H, H)), full((1, H)),
        ],
        out_specs=full((1, H)),
        out_shape=jax.ShapeDtypeStruct((1, H), _f32),
    )(g1, a1, g2, a2, g3, a3, w1T, b1, w2T, b2, w3T, b3)


def kernel(x, edge_index, edge_weight, indices, W1l, b1l, W1r, W2l, b2l, W2r,
           W3l, b3l, W3r, Wlin1, blin1, Wlin2, blin2, Wlin3, blin3):
    src = edge_index[0]
    dst = edge_index[1]
    ew = edge_weight
    z16 = jnp.zeros((N_PAD, D_IN), _f32)
    z32 = jnp.zeros((N_PAD, 32), _f32)
    ones_rows = jnp.ones((BATCH, D_IN), _f32)

    w1lT = W1l.T
    w1rT = W1r.T
    w2lT = W2l.T
    w2rT = W2r.T
    w3lT = W3l.T
    w3rT = W3r.T
    wlin1T = Wlin1.T
    wlin2T = jnp.zeros((H, H), _f32).at[:, :64].set(Wlin2.T)
    b2pad = jnp.zeros((1, H), _f32).at[:, :64].set(blin2[None, :])
    wlin3T = jnp.zeros((H, H), _f32).at[:64, :2].set(Wlin3.T)
    b3row = jnp.zeros((1, H), _f32).at[:, :2].set(blin3[None, :])

    sc_layer1, sc_agg_h = _get_sc_kernels()
    aggp, cntp = sc_layer1(x, src, dst, ew, z16, ones_rows)
    h10, h11, h12, h13, g1, a1 = _tc_b1(aggp, cntp, x, w1lT, b1l[None, :],
                                        w1rT)
    agg2 = sc_agg_h(h10, h11, h12, h13, src, dst, ew, z32)
    h20, h21, h22, h23, g2, a2 = _tc_mid(agg2, cntp, (h10, h11, h12, h13),
                                         w2lT, b2l[None, :], w2rT, True)
    agg3 = sc_agg_h(h20, h21, h22, h23, src, dst, ew, z32)
    g3, a3 = _tc_mid(agg3, cntp, (h20, h21, h22, h23), w3lT, b3l[None, :],
                     w3rT, False)

    out = _tc_head(g1, a1, g2, a2, g3, a3, wlin1T, blin1[None, :],
                   wlin2T, b2pad, wlin3T, b3row)
    sel = jnp.take(out[:, :2], indices, axis=0)
    return sel
